# Initial kernel scaffold; baseline (speedup 1.0000x reference)
import jax
import jax.numpy as jnp
from jax.experimental import pallas as pl


def kernel(s_cat, k_cat, k_cont, o_cont, target, edge_index, params):
    raise NotImplementedError("write your pallas kernel here")


# speedup vs baseline: 1.8106x; 1.8106x over previous
import functools

import jax
import jax.numpy as jnp
from jax import lax
from jax.experimental import pallas as pl
from jax.experimental.pallas import tpu as pltpu
from jax.experimental.pallas import tpu_sc as plsc

_N = 10000
_E = 320000
_T = 16
_ENC = 12
_H = 128
_NL = 2

_NC = 2
_NS = 16
_NW = _NC * _NS
_NP = 10240
_K = 128
_NCHUNK = 80
_EW = _K * _NCHUNK
_EPAD = _NW * _EW
_RPT = _NP // _NS

_BN = 512
_GRID = _NP // _BN

_f32 = jnp.float32


def _spmm_body(xs, gidx, sidx, out, gv, sv, rows, zbuf, agg, sem):
    cid = lax.axis_index("c")
    tid = lax.axis_index("s")
    wid = cid * _NS + tid
    pltpu.sync_copy(gidx.at[wid], gv)
    pltpu.sync_copy(sidx.at[wid], sv)

    zero16 = jnp.zeros((16,), _f32)

    def _zb(r, c):
        for k in range(_H // 16):
            zbuf[r, pl.ds(k * 16, 16)] = zero16
        return c

    lax.fori_loop(0, 64, _zb, 0)
    base = tid * _RPT

    def _zs(j, c):
        pltpu.sync_copy(zbuf, agg.at[pl.ds(base + j * 64, 64)])
        return c

    lax.fori_loop(0, _RPT // 64, _zs, 0)
    plsc.subcore_barrier()

    def _chunk(j, c):
        pltpu.async_copy(xs.at[gv.at[j]], rows, sem).wait()
        pltpu.sync_copy(rows, agg.at[sv.at[j]], add=True)
        return c

    lax.fori_loop(0, _NCHUNK, _chunk, 0)
    plsc.subcore_barrier()

    def _fl(j, c):
        off = base + j * _K
        pltpu.sync_copy(agg.at[pl.ds(off, _K)], rows)
        pltpu.sync_copy(rows, out.at[cid, pl.ds(off, _K)])
        return c

    lax.fori_loop(0, _RPT // _K, _fl, 0)


@functools.lru_cache(maxsize=None)
def _get_spmm():
    return pl.kernel(
        _spmm_body,
        out_type=jax.ShapeDtypeStruct((_NC, _NP, _H), _f32),
        mesh=plsc.VectorSubcoreMesh(
            core_axis_name="c", subcore_axis_name="s",
            num_cores=_NC, num_subcores=_NS),
        scratch_types=[
            pltpu.VMEM((_NCHUNK, _K), jnp.int32),
            pltpu.VMEM((_NCHUNK, _K), jnp.int32),
            pltpu.VMEM((_K, _H), _f32),
            pltpu.VMEM((64, _H), _f32),
            pltpu.VMEM_SHARED((_NP, _H), _f32),
            pltpu.SemaphoreType.DMA,
        ],
    )


def _row_mask(pid):
    nid = lax.broadcasted_iota(jnp.int32, (_BN, 1), 0) + pid * _BN
    return (nid < _N).astype(_f32)


def _embed_kernel(s0, kc, kc0, kc1, oc0, oc1, tg, dgo, dgi,
                  emb0, emb1, sw, sb, kct, kcv, kcb, ocv, ocb, tv, tb,
                  hdw, hdb, fdw, fdb,
                  xs_hist, xs_fut, h0, hs0, norms):
    pid = pl.program_id(0)
    mask = _row_mask(pid)
    dout = jnp.maximum(dgo[0, :, 0:1] + dgo[1, :, 0:1], 1.0)
    din = jnp.maximum(dgi[0, :, 0:1] + dgi[1, :, 0:1], 1.0)
    no = lax.rsqrt(dout)
    ni = lax.rsqrt(din)
    norms[:, 0:1] = no
    norms[:, 1:2] = ni

    i100 = lax.broadcasted_iota(jnp.int32, (_BN, 100), 1)
    e0 = jnp.dot((s0[:, 0:1] == i100).astype(_f32), emb0[...])
    e1 = jnp.dot((s0[:, 1:2] == i100).astype(_f32), emb1[...])
    init = jnp.dot(jnp.concatenate([e0, e1], axis=1), sw[...]) + sb[...]
    for l in range(_NL):
        hl = init[:, l * _H:(l + 1) * _H] * mask
        h0[l] = hl
        hs0[l] = hl * no

    i50 = lax.broadcasted_iota(jnp.int32, (_BN, 50), 1)
    for t in range(_T):
        ekc = jnp.dot((kc[:, t:t + 1] == i50).astype(_f32), kct[...])
        known = [ekc,
                 kc0[:, t:t + 1] * kcv[0:1] + kcb[0:1],
                 kc1[:, t:t + 1] * kcv[1:2] + kcb[1:2]]
        if t < _ENC:
            feat = jnp.concatenate(
                known + [oc0[:, t:t + 1] * ocv[0:1] + ocb[0:1],
                         oc1[:, t:t + 1] * ocv[1:2] + ocb[1:2],
                         tg[:, t:t + 1] * tv[...] + tb[...]], axis=1)
            v = jnp.dot(feat, hdw[...]) + hdb[...]
            xs_hist[t] = v * no * mask
        else:
            feat = jnp.concatenate(known, axis=1)
            v = jnp.dot(feat, fdw[...]) + fdb[...]
            xs_fut[t - _ENC] = v * no * mask


def _step_kernel(ax, ah, hprev, norms, wi, bi, wh, bh, h_out, hs_out):
    pid = pl.program_id(0)
    mask = _row_mask(pid)
    no = norms[:, 0:1]
    ni = norms[:, 1:2]
    axn = (ax[0] + ax[1]) * ni
    ahn = (ah[0] + ah[1]) * ni
    i3 = jnp.dot(axn, wi[...]) + bi[...]
    h3 = jnp.dot(ahn, wh[...]) + bh[...]
    r = jax.nn.sigmoid(i3[:, :_H] + h3[:, :_H])
    z = jax.nn.sigmoid(i3[:, _H:2 * _H] + h3[:, _H:2 * _H])
    n = jnp.tanh(i3[:, 2 * _H:] + r * h3[:, 2 * _H:])
    h = ((1.0 - z) * n + z * hprev[...]) * mask
    h_out[...] = h
    hs_out[...] = h * no


def _proj_kernel(hseq, w, b, y):
    cols = [jnp.dot(hseq[t], w[...]) + b[...]
            for t in range(_T - _ENC)]
    y[...] = jnp.concatenate(cols, axis=1)


def _full(shape):
    return pl.BlockSpec(shape, lambda i: (0,) * len(shape))


def _nblk(*lead):
    nax = len(lead)
    shape = lead + (_BN, _H)
    idx = lambda i: (0,) * nax + (i, 0)
    return pl.BlockSpec(shape, idx)


def _make_tc_kernels(interpret=False):
    embed = pl.pallas_call(
        _embed_kernel,
        grid=(_GRID,),
        in_specs=[
            pl.BlockSpec((_BN, 2), lambda i: (i, 0)),
            pl.BlockSpec((_BN, _T), lambda i: (i, 0)),
            pl.BlockSpec((_BN, _T), lambda i: (i, 0)),
            pl.BlockSpec((_BN, _T), lambda i: (i, 0)),
            pl.BlockSpec((_BN, _T), lambda i: (i, 0)),
            pl.BlockSpec((_BN, _T), lambda i: (i, 0)),
            pl.BlockSpec((_BN, _T), lambda i: (i, 0)),
            _nblk(_NC),
            _nblk(_NC),
            _full((100, _H)),
            _full((100, _H)),
            _full((2 * _H, 2 * _H)),
            _full((1, 2 * _H)),
            _full((50, _H)),
            _full((2, _H)),
            _full((2, _H)),
            _full((2, _H)),
            _full((2, _H)),
            _full((1, _H)),
            _full((1, _H)),
            _full((6 * _H, _H)),
            _full((1, _H)),
            _full((3 * _H, _H)),
            _full((1, _H)),
        ],
        out_specs=[
            _nblk(_ENC),
            _nblk(_T - _ENC),
            _nblk(_NL),
            _nblk(_NL),
            pl.BlockSpec((_BN, 2), lambda i: (i, 0)),
        ],
        out_shape=[
            jax.ShapeDtypeStruct((_ENC, _NP, _H), _f32),
            jax.ShapeDtypeStruct((_T - _ENC, _NP, _H), _f32),
            jax.ShapeDtypeStruct((_NL, _NP, _H), _f32),
            jax.ShapeDtypeStruct((_NL, _NP, _H), _f32),
            jax.ShapeDtypeStruct((_NP, 2), _f32),
        ],
        interpret=interpret,
    )

    step = pl.pallas_call(
        _step_kernel,
        grid=(_GRID,),
        in_specs=[
            _nblk(_NC),
            _nblk(_NC),
            _nblk(),
            pl.BlockSpec((_BN, 2), lambda i: (i, 0)),
            _full((_H, 3 * _H)),
            _full((1, 3 * _H)),
            _full((_H, 3 * _H)),
            _full((1, 3 * _H)),
        ],
        out_specs=[_nblk(), _nblk()],
        out_shape=[
            jax.ShapeDtypeStruct((_NP, _H), _f32),
            jax.ShapeDtypeStruct((_NP, _H), _f32),
        ],
        interpret=interpret,
    )

    proj = pl.pallas_call(
        _proj_kernel,
        grid=(_GRID,),
        in_specs=[
            _nblk(_T - _ENC),
            _full((_H, 1)),
            _full((1, 1)),
        ],
        out_specs=pl.BlockSpec((_BN, _T - _ENC), lambda i: (i, 0)),
        out_shape=jax.ShapeDtypeStruct((_NP, _T - _ENC), _f32),
        interpret=interpret,
    )
    return embed, step, proj


_EMBED, _STEP, _PROJ = _make_tc_kernels()


def _pad_nodes(a):
    return jnp.pad(a, ((0, _NP - _N),) + ((0, 0),) * (a.ndim - 1))


def kernel(s_cat, k_cat, k_cont, o_cont, target, edge_index, params):
    spmm = _get_spmm()
    p = params

    s0 = _pad_nodes(s_cat[:, 0, :].astype(jnp.int32))
    kc = _pad_nodes(k_cat[:, :, 0].astype(jnp.int32))
    kc0 = _pad_nodes(k_cont[:, :, 0])
    kc1 = _pad_nodes(k_cont[:, :, 1])
    oc0 = _pad_nodes(o_cont[:, :, 0])
    oc1 = _pad_nodes(o_cont[:, :, 1])
    tg = _pad_nodes(target[:, :, 0])
    src = jnp.pad(edge_index[0], (0, _EPAD - _E),
                  constant_values=_NP - 1).astype(jnp.int32)
    dst = jnp.pad(edge_index[1], (0, _EPAD - _E),
                  constant_values=_NP - 1).astype(jnp.int32)
    src_r = src.reshape(_NW, _NCHUNK, _K)
    dst_r = dst.reshape(_NW, _NCHUNK, _K)

    ones_x = jnp.ones((_NP, _H), _f32)
    dgo = spmm(ones_x, dst_r, src_r)
    dgi = spmm(ones_x, src_r, dst_r)

    xs_hist, xs_fut, h0, hs0, norms = _EMBED(
        s0, kc, kc0, kc1, oc0, oc1, tg, dgo, dgi,
        p["s_cat_emb"][0], p["s_cat_emb"][1],
        p["static_W"], p["static_b"][None, :],
        p["k_cat_emb"][0],
        p["k_cont_vec"], p["k_cont_bias"],
        p["o_cont_vec"], p["o_cont_bias"],
        p["tgt_vec"], p["tgt_bias"],
        p["hist_down_W"], p["hist_down_b"][None, :],
        p["fut_down_W"], p["fut_down_b"][None, :])

    def run_layer(xs_list, h, hs, lp):
        wi, bi = lp["Wi"], lp["bi"][None, :]
        wh, bh = lp["Wh"], lp["bh"][None, :]
        ax_list = [spmm(x, src_r, dst_r) for x in xs_list]
        outs = []
        for ax in ax_list:
            ah = spmm(hs, src_r, dst_r)
            h, hs = _STEP(ax, ah, h, norms, wi, bi, wh, bh)
            outs.append((h, hs))
        return outs

    hist0 = run_layer([xs_hist[t] for t in range(_ENC)],
                      h0[0], hs0[0], p["hist_layers"][0])
    hist1 = run_layer([hs for _, hs in hist0],
                      h0[1], hs0[1], p["hist_layers"][1])
    fut0 = run_layer([xs_fut[t] for t in range(_T - _ENC)],
                     hist0[-1][0], hist0[-1][1], p["fut_layers"][0])
    fut1 = run_layer([hs for _, hs in fut0],
                     hist1[-1][0], hist1[-1][1], p["fut_layers"][1])

    hseq = jnp.stack([h for h, _ in fut1])
    y = _PROJ(hseq, p["out_W"], p["out_b"][None, :])
    return y[:_N, :, None]


# speedup vs baseline: 1.9808x; 1.0940x over previous
import functools

import jax
import jax.numpy as jnp
from jax import lax
from jax.experimental import pallas as pl
from jax.experimental.pallas import tpu as pltpu
from jax.experimental.pallas import tpu_sc as plsc

_N = 10000
_E = 320000
_T = 16
_ENC = 12
_H = 128
_NL = 2

_NC = 2
_NS = 16
_NW = _NC * _NS
_NP = 10240
_K = 128
_NCHUNK = 80
_EW = _K * _NCHUNK
_EPAD = _NW * _EW
_RPT = _NP // _NS

_BN = 512
_GRID = _NP // _BN

_f32 = jnp.float32


_NBUF = 2
_NPH = 2
_CPP = _NCHUNK // _NPH
_GPP = _CPP // _NBUF


def _spmm_body(xs, gidx, sidx, out, gv, sv, rows, zbuf, agg, gsems, ssems):
    cid = lax.axis_index("c")
    tid = lax.axis_index("s")
    wid = cid * _NS + tid

    zero16 = jnp.zeros((16,), _f32)
    for r in range(16):
        for k in range(_H // 16):
            zbuf[r, pl.ds(k * 16, 16)] = zero16
    base = tid * _RPT

    def _zs(j, c):
        pltpu.sync_copy(zbuf, agg.at[pl.ds(base + j * 16, 16)])
        return c

    lax.fori_loop(0, _RPT // 16, _zs, 0)
    plsc.subcore_barrier()

    def _gather(j, b):
        pltpu.async_copy(xs.at[gv.at[j]], rows.at[b], gsems.at[b])

    def _wait_gather(j, b):
        pltpu.make_async_copy(xs.at[gv.at[j]], rows.at[b], gsems.at[b]).wait()

    def _scatter(j, b):
        pltpu.async_copy(rows.at[b], agg.at[sv.at[j]], ssems.at[b], add=True)

    def _wait_scatter(j, b):
        pltpu.make_async_copy(rows.at[b], agg.at[sv.at[j]],
                              ssems.at[b]).wait()

    for ph in range(_NPH):
        pltpu.sync_copy(gidx.at[wid, pl.ds(ph * _CPP, _CPP)], gv)
        pltpu.sync_copy(sidx.at[wid, pl.ds(ph * _CPP, _CPP)], sv)
        for b in range(_NBUF):
            _gather(b, b)

        def _grp(g, c):
            jb = g * _NBUF
            for b in range(_NBUF):
                _wait_gather(jb + b, b)
                _scatter(jb + b, b)
            for b in range(_NBUF):
                _wait_scatter(jb + b, b)
                @pl.when(g < _GPP - 1)
                def _():
                    _gather(jb + _NBUF + b, b)
            return c

        lax.fori_loop(0, _GPP, _grp, 0)

    plsc.subcore_barrier()

    def _fl(j, c):
        off = base + j * _K
        pltpu.sync_copy(agg.at[pl.ds(off, _K)], rows.at[0])
        pltpu.sync_copy(rows.at[0], out.at[cid, pl.ds(off, _K)])
        return c

    lax.fori_loop(0, _RPT // _K, _fl, 0)


@functools.lru_cache(maxsize=None)
def _get_spmm():
    return pl.kernel(
        _spmm_body,
        out_type=jax.ShapeDtypeStruct((_NC, _NP, _H), _f32),
        mesh=plsc.VectorSubcoreMesh(
            core_axis_name="c", subcore_axis_name="s",
            num_cores=_NC, num_subcores=_NS),
        scratch_types=[
            pltpu.VMEM((_CPP, _K), jnp.int32),
            pltpu.VMEM((_CPP, _K), jnp.int32),
            pltpu.VMEM((_NBUF, _K, _H), _f32),
            pltpu.VMEM((16, _H), _f32),
            pltpu.VMEM_SHARED((_NP, _H), _f32),
            pltpu.SemaphoreType.DMA((_NBUF,)),
            pltpu.SemaphoreType.DMA((_NBUF,)),
        ],
    )


def _row_mask(pid):
    nid = lax.broadcasted_iota(jnp.int32, (_BN, 1), 0) + pid * _BN
    return (nid < _N).astype(_f32)


def _embed_kernel(s0, kc, kc0, kc1, oc0, oc1, tg, dgo, dgi,
                  emb0, emb1, sw, sb, kct, kcv, kcb, ocv, ocb, tv, tb,
                  hdw, hdb, fdw, fdb,
                  xs_hist, xs_fut, h0, hs0, norms):
    pid = pl.program_id(0)
    mask = _row_mask(pid)
    dout = jnp.maximum(dgo[0, :, 0:1] + dgo[1, :, 0:1], 1.0)
    din = jnp.maximum(dgi[0, :, 0:1] + dgi[1, :, 0:1], 1.0)
    no = lax.rsqrt(dout)
    ni = lax.rsqrt(din)
    norms[:, 0:1] = no
    norms[:, 1:2] = ni

    i100 = lax.broadcasted_iota(jnp.int32, (_BN, 100), 1)
    e0 = jnp.dot((s0[:, 0:1] == i100).astype(_f32), emb0[...])
    e1 = jnp.dot((s0[:, 1:2] == i100).astype(_f32), emb1[...])
    init = jnp.dot(jnp.concatenate([e0, e1], axis=1), sw[...]) + sb[...]
    for l in range(_NL):
        hl = init[:, l * _H:(l + 1) * _H] * mask
        h0[l] = hl
        hs0[l] = hl * no

    i50 = lax.broadcasted_iota(jnp.int32, (_BN, 50), 1)
    for t in range(_T):
        ekc = jnp.dot((kc[:, t:t + 1] == i50).astype(_f32), kct[...])
        known = [ekc,
                 kc0[:, t:t + 1] * kcv[0:1] + kcb[0:1],
                 kc1[:, t:t + 1] * kcv[1:2] + kcb[1:2]]
        if t < _ENC:
            feat = jnp.concatenate(
                known + [oc0[:, t:t + 1] * ocv[0:1] + ocb[0:1],
                         oc1[:, t:t + 1] * ocv[1:2] + ocb[1:2],
                         tg[:, t:t + 1] * tv[...] + tb[...]], axis=1)
            v = jnp.dot(feat, hdw[...]) + hdb[...]
            xs_hist[t] = v * no * mask
        else:
            feat = jnp.concatenate(known, axis=1)
            v = jnp.dot(feat, fdw[...]) + fdb[...]
            xs_fut[t - _ENC] = v * no * mask


def _step_kernel(ax, ah, hprev, norms, wi, bi, wh, bh, h_out, hs_out):
    pid = pl.program_id(0)
    mask = _row_mask(pid)
    no = norms[:, 0:1]
    ni = norms[:, 1:2]
    axn = (ax[0] + ax[1]) * ni
    ahn = (ah[0] + ah[1]) * ni
    i3 = jnp.dot(axn, wi[...]) + bi[...]
    h3 = jnp.dot(ahn, wh[...]) + bh[...]
    r = jax.nn.sigmoid(i3[:, :_H] + h3[:, :_H])
    z = jax.nn.sigmoid(i3[:, _H:2 * _H] + h3[:, _H:2 * _H])
    n = jnp.tanh(i3[:, 2 * _H:] + r * h3[:, 2 * _H:])
    h = ((1.0 - z) * n + z * hprev[...]) * mask
    h_out[...] = h
    hs_out[...] = h * no


def _proj_kernel(hseq, w, b, y):
    cols = [jnp.dot(hseq[t], w[...]) + b[...]
            for t in range(_T - _ENC)]
    y[...] = jnp.concatenate(cols, axis=1)


def _full(shape):
    return pl.BlockSpec(shape, lambda i: (0,) * len(shape))


def _nblk(*lead):
    nax = len(lead)
    shape = lead + (_BN, _H)
    idx = lambda i: (0,) * nax + (i, 0)
    return pl.BlockSpec(shape, idx)


def _make_tc_kernels(interpret=False):
    embed = pl.pallas_call(
        _embed_kernel,
        grid=(_GRID,),
        in_specs=[
            pl.BlockSpec((_BN, 2), lambda i: (i, 0)),
            pl.BlockSpec((_BN, _T), lambda i: (i, 0)),
            pl.BlockSpec((_BN, _T), lambda i: (i, 0)),
            pl.BlockSpec((_BN, _T), lambda i: (i, 0)),
            pl.BlockSpec((_BN, _T), lambda i: (i, 0)),
            pl.BlockSpec((_BN, _T), lambda i: (i, 0)),
            pl.BlockSpec((_BN, _T), lambda i: (i, 0)),
            _nblk(_NC),
            _nblk(_NC),
            _full((100, _H)),
            _full((100, _H)),
            _full((2 * _H, 2 * _H)),
            _full((1, 2 * _H)),
            _full((50, _H)),
            _full((2, _H)),
            _full((2, _H)),
            _full((2, _H)),
            _full((2, _H)),
            _full((1, _H)),
            _full((1, _H)),
            _full((6 * _H, _H)),
            _full((1, _H)),
            _full((3 * _H, _H)),
            _full((1, _H)),
        ],
        out_specs=[
            _nblk(_ENC),
            _nblk(_T - _ENC),
            _nblk(_NL),
            _nblk(_NL),
            pl.BlockSpec((_BN, 2), lambda i: (i, 0)),
        ],
        out_shape=[
            jax.ShapeDtypeStruct((_ENC, _NP, _H), _f32),
            jax.ShapeDtypeStruct((_T - _ENC, _NP, _H), _f32),
            jax.ShapeDtypeStruct((_NL, _NP, _H), _f32),
            jax.ShapeDtypeStruct((_NL, _NP, _H), _f32),
            jax.ShapeDtypeStruct((_NP, 2), _f32),
        ],
        interpret=interpret,
    )

    step = pl.pallas_call(
        _step_kernel,
        grid=(_GRID,),
        in_specs=[
            _nblk(_NC),
            _nblk(_NC),
            _nblk(),
            pl.BlockSpec((_BN, 2), lambda i: (i, 0)),
            _full((_H, 3 * _H)),
            _full((1, 3 * _H)),
            _full((_H, 3 * _H)),
            _full((1, 3 * _H)),
        ],
        out_specs=[_nblk(), _nblk()],
        out_shape=[
            jax.ShapeDtypeStruct((_NP, _H), _f32),
            jax.ShapeDtypeStruct((_NP, _H), _f32),
        ],
        interpret=interpret,
    )

    proj = pl.pallas_call(
        _proj_kernel,
        grid=(_GRID,),
        in_specs=[
            _nblk(_T - _ENC),
            _full((_H, 1)),
            _full((1, 1)),
        ],
        out_specs=pl.BlockSpec((_BN, _T - _ENC), lambda i: (i, 0)),
        out_shape=jax.ShapeDtypeStruct((_NP, _T - _ENC), _f32),
        interpret=interpret,
    )
    return embed, step, proj


_EMBED, _STEP, _PROJ = _make_tc_kernels()


def _pad_nodes(a):
    return jnp.pad(a, ((0, _NP - _N),) + ((0, 0),) * (a.ndim - 1))


def kernel(s_cat, k_cat, k_cont, o_cont, target, edge_index, params):
    spmm = _get_spmm()
    p = params

    s0 = _pad_nodes(s_cat[:, 0, :].astype(jnp.int32))
    kc = _pad_nodes(k_cat[:, :, 0].astype(jnp.int32))
    kc0 = _pad_nodes(k_cont[:, :, 0])
    kc1 = _pad_nodes(k_cont[:, :, 1])
    oc0 = _pad_nodes(o_cont[:, :, 0])
    oc1 = _pad_nodes(o_cont[:, :, 1])
    tg = _pad_nodes(target[:, :, 0])
    src = jnp.pad(edge_index[0], (0, _EPAD - _E),
                  constant_values=_NP - 1).astype(jnp.int32)
    dst = jnp.pad(edge_index[1], (0, _EPAD - _E),
                  constant_values=_NP - 1).astype(jnp.int32)
    src_r = src.reshape(_NW, _NCHUNK, _K)
    dst_r = dst.reshape(_NW, _NCHUNK, _K)

    ones_x = jnp.ones((_NP, _H), _f32)
    dgo = spmm(ones_x, dst_r, src_r)
    dgi = spmm(ones_x, src_r, dst_r)

    xs_hist, xs_fut, h0, hs0, norms = _EMBED(
        s0, kc, kc0, kc1, oc0, oc1, tg, dgo, dgi,
        p["s_cat_emb"][0], p["s_cat_emb"][1],
        p["static_W"], p["static_b"][None, :],
        p["k_cat_emb"][0],
        p["k_cont_vec"], p["k_cont_bias"],
        p["o_cont_vec"], p["o_cont_bias"],
        p["tgt_vec"], p["tgt_bias"],
        p["hist_down_W"], p["hist_down_b"][None, :],
        p["fut_down_W"], p["fut_down_b"][None, :])

    def run_layer(xs_list, h, hs, lp):
        wi, bi = lp["Wi"], lp["bi"][None, :]
        wh, bh = lp["Wh"], lp["bh"][None, :]
        ax_list = [spmm(x, src_r, dst_r) for x in xs_list]
        outs = []
        for ax in ax_list:
            ah = spmm(hs, src_r, dst_r)
            h, hs = _STEP(ax, ah, h, norms, wi, bi, wh, bh)
            outs.append((h, hs))
        return outs

    hist0 = run_layer([xs_hist[t] for t in range(_ENC)],
                      h0[0], hs0[0], p["hist_layers"][0])
    hist1 = run_layer([hs for _, hs in hist0],
                      h0[1], hs0[1], p["hist_layers"][1])
    fut0 = run_layer([xs_fut[t] for t in range(_T - _ENC)],
                     hist0[-1][0], hist0[-1][1], p["fut_layers"][0])
    fut1 = run_layer([hs for _, hs in fut0],
                     hist1[-1][0], hist1[-1][1], p["fut_layers"][1])

    hseq = jnp.stack([h for h, _ in fut1])
    y = _PROJ(hseq, p["out_W"], p["out_b"][None, :])
    return y[:_N, :, None]


# speedup vs baseline: 1.9825x; 1.0009x over previous
import functools

import jax
import jax.numpy as jnp
from jax import lax
from jax.experimental import pallas as pl
from jax.experimental.pallas import tpu as pltpu
from jax.experimental.pallas import tpu_sc as plsc

_N = 10000
_E = 320000
_T = 16
_ENC = 12
_H = 128
_NL = 2

_NC = 2
_NS = 16
_NW = _NC * _NS
_NP = 10240
_K = 128
_NCHUNK = 80
_EW = _K * _NCHUNK
_EPAD = _NW * _EW
_RPT = _NP // _NS

_BN = 1024
_GRID = _NP // _BN

_f32 = jnp.float32


_NBUF = 2
_NPH = 2
_CPP = _NCHUNK // _NPH
_GPP = _CPP // _NBUF


def _spmm_body(xs, gidx, sidx, out, gv, sv, rows, zbuf, agg, gsems, ssems):
    cid = lax.axis_index("c")
    tid = lax.axis_index("s")
    wid = cid * _NS + tid

    zero16 = jnp.zeros((16,), _f32)
    for r in range(16):
        for k in range(_H // 16):
            zbuf[r, pl.ds(k * 16, 16)] = zero16
    base = tid * _RPT

    def _zs(j, c):
        pltpu.sync_copy(zbuf, agg.at[pl.ds(base + j * 16, 16)])
        return c

    lax.fori_loop(0, _RPT // 16, _zs, 0)
    plsc.subcore_barrier()

    def _gather(j, b):
        pltpu.async_copy(xs.at[gv.at[j]], rows.at[b], gsems.at[b])

    def _wait_gather(j, b):
        pltpu.make_async_copy(xs.at[gv.at[j]], rows.at[b], gsems.at[b]).wait()

    def _scatter(j, b):
        pltpu.async_copy(rows.at[b], agg.at[sv.at[j]], ssems.at[b], add=True)

    def _wait_scatter(j, b):
        pltpu.make_async_copy(rows.at[b], agg.at[sv.at[j]],
                              ssems.at[b]).wait()

    for ph in range(_NPH):
        pltpu.sync_copy(gidx.at[wid, pl.ds(ph * _CPP, _CPP)], gv)
        pltpu.sync_copy(sidx.at[wid, pl.ds(ph * _CPP, _CPP)], sv)
        for b in range(_NBUF):
            _gather(b, b)

        def _grp(g, c):
            jb = g * _NBUF
            for b in range(_NBUF):
                _wait_gather(jb + b, b)
                _scatter(jb + b, b)
            for b in range(_NBUF):
                _wait_scatter(jb + b, b)
                @pl.when(g < _GPP - 1)
                def _():
                    _gather(jb + _NBUF + b, b)
            return c

        lax.fori_loop(0, _GPP, _grp, 0)

    plsc.subcore_barrier()

    def _fl(j, c):
        off = base + j * _K
        pltpu.sync_copy(agg.at[pl.ds(off, _K)], rows.at[0])
        pltpu.sync_copy(rows.at[0], out.at[cid, pl.ds(off, _K)])
        return c

    lax.fori_loop(0, _RPT // _K, _fl, 0)


@functools.lru_cache(maxsize=None)
def _get_spmm():
    return pl.kernel(
        _spmm_body,
        out_type=jax.ShapeDtypeStruct((_NC, _NP, _H), _f32),
        mesh=plsc.VectorSubcoreMesh(
            core_axis_name="c", subcore_axis_name="s",
            num_cores=_NC, num_subcores=_NS),
        scratch_types=[
            pltpu.VMEM((_CPP, _K), jnp.int32),
            pltpu.VMEM((_CPP, _K), jnp.int32),
            pltpu.VMEM((_NBUF, _K, _H), _f32),
            pltpu.VMEM((16, _H), _f32),
            pltpu.VMEM_SHARED((_NP, _H), _f32),
            pltpu.SemaphoreType.DMA((_NBUF,)),
            pltpu.SemaphoreType.DMA((_NBUF,)),
        ],
    )


def _row_mask(pid):
    nid = lax.broadcasted_iota(jnp.int32, (_BN, 1), 0) + pid * _BN
    return (nid < _N).astype(_f32)


def _embed_kernel(s0, kc, kc0, kc1, oc0, oc1, tg, dgo, dgi,
                  emb0, emb1, sw, sb, kct, kcv, kcb, ocv, ocb, tv, tb,
                  hdw, hdb, fdw, fdb,
                  xs_hist, xs_fut, h0, hs0, norms):
    pid = pl.program_id(0)
    mask = _row_mask(pid)
    dout = jnp.maximum(dgo[0, :, 0:1] + dgo[1, :, 0:1], 1.0)
    din = jnp.maximum(dgi[0, :, 0:1] + dgi[1, :, 0:1], 1.0)
    no = lax.rsqrt(dout)
    ni = lax.rsqrt(din)
    norms[:, 0:1] = no
    norms[:, 1:2] = ni

    i100 = lax.broadcasted_iota(jnp.int32, (_BN, 100), 1)
    e0 = jnp.dot((s0[:, 0:1] == i100).astype(_f32), emb0[...])
    e1 = jnp.dot((s0[:, 1:2] == i100).astype(_f32), emb1[...])
    init = jnp.dot(jnp.concatenate([e0, e1], axis=1), sw[...]) + sb[...]
    for l in range(_NL):
        hl = init[:, l * _H:(l + 1) * _H] * mask
        h0[l] = hl
        hs0[l] = hl * no

    i50 = lax.broadcasted_iota(jnp.int32, (_BN, 50), 1)
    for t in range(_T):
        ekc = jnp.dot((kc[:, t:t + 1] == i50).astype(_f32), kct[...])
        known = [ekc,
                 kc0[:, t:t + 1] * kcv[0:1] + kcb[0:1],
                 kc1[:, t:t + 1] * kcv[1:2] + kcb[1:2]]
        if t < _ENC:
            feat = jnp.concatenate(
                known + [oc0[:, t:t + 1] * ocv[0:1] + ocb[0:1],
                         oc1[:, t:t + 1] * ocv[1:2] + ocb[1:2],
                         tg[:, t:t + 1] * tv[...] + tb[...]], axis=1)
            v = jnp.dot(feat, hdw[...]) + hdb[...]
            xs_hist[t] = v * no * mask
        else:
            feat = jnp.concatenate(known, axis=1)
            v = jnp.dot(feat, fdw[...]) + fdb[...]
            xs_fut[t - _ENC] = v * no * mask


def _step_kernel(ax, ah, hprev, norms, wi, bi, wh, bh, h_out, hs_out):
    pid = pl.program_id(0)
    mask = _row_mask(pid)
    no = norms[:, 0:1]
    ni = norms[:, 1:2]
    axn = (ax[0] + ax[1]) * ni
    ahn = (ah[0] + ah[1]) * ni
    i3 = jnp.dot(axn, wi[...]) + bi[...]
    h3 = jnp.dot(ahn, wh[...]) + bh[...]
    r = jax.nn.sigmoid(i3[:, :_H] + h3[:, :_H])
    z = jax.nn.sigmoid(i3[:, _H:2 * _H] + h3[:, _H:2 * _H])
    n = jnp.tanh(i3[:, 2 * _H:] + r * h3[:, 2 * _H:])
    h = ((1.0 - z) * n + z * hprev[...]) * mask
    h_out[...] = h
    hs_out[...] = h * no


def _proj_kernel(hseq, w, b, y):
    cols = [jnp.dot(hseq[t], w[...]) + b[...]
            for t in range(_T - _ENC)]
    y[...] = jnp.concatenate(cols, axis=1)


def _full(shape):
    return pl.BlockSpec(shape, lambda i: (0,) * len(shape))


def _nblk(*lead):
    nax = len(lead)
    shape = lead + (_BN, _H)
    idx = lambda i: (0,) * nax + (i, 0)
    return pl.BlockSpec(shape, idx)


def _make_tc_kernels(interpret=False):
    embed = pl.pallas_call(
        _embed_kernel,
        grid=(_GRID,),
        in_specs=[
            pl.BlockSpec((_BN, 2), lambda i: (i, 0)),
            pl.BlockSpec((_BN, _T), lambda i: (i, 0)),
            pl.BlockSpec((_BN, _T), lambda i: (i, 0)),
            pl.BlockSpec((_BN, _T), lambda i: (i, 0)),
            pl.BlockSpec((_BN, _T), lambda i: (i, 0)),
            pl.BlockSpec((_BN, _T), lambda i: (i, 0)),
            pl.BlockSpec((_BN, _T), lambda i: (i, 0)),
            _nblk(_NC),
            _nblk(_NC),
            _full((100, _H)),
            _full((100, _H)),
            _full((2 * _H, 2 * _H)),
            _full((1, 2 * _H)),
            _full((50, _H)),
            _full((2, _H)),
            _full((2, _H)),
            _full((2, _H)),
            _full((2, _H)),
            _full((1, _H)),
            _full((1, _H)),
            _full((6 * _H, _H)),
            _full((1, _H)),
            _full((3 * _H, _H)),
            _full((1, _H)),
        ],
        out_specs=[
            _nblk(_ENC),
            _nblk(_T - _ENC),
            _nblk(_NL),
            _nblk(_NL),
            pl.BlockSpec((_BN, 2), lambda i: (i, 0)),
        ],
        out_shape=[
            jax.ShapeDtypeStruct((_ENC, _NP, _H), _f32),
            jax.ShapeDtypeStruct((_T - _ENC, _NP, _H), _f32),
            jax.ShapeDtypeStruct((_NL, _NP, _H), _f32),
            jax.ShapeDtypeStruct((_NL, _NP, _H), _f32),
            jax.ShapeDtypeStruct((_NP, 2), _f32),
        ],
        interpret=interpret,
    )

    step = pl.pallas_call(
        _step_kernel,
        grid=(_GRID,),
        in_specs=[
            _nblk(_NC),
            _nblk(_NC),
            _nblk(),
            pl.BlockSpec((_BN, 2), lambda i: (i, 0)),
            _full((_H, 3 * _H)),
            _full((1, 3 * _H)),
            _full((_H, 3 * _H)),
            _full((1, 3 * _H)),
        ],
        out_specs=[_nblk(), _nblk()],
        out_shape=[
            jax.ShapeDtypeStruct((_NP, _H), _f32),
            jax.ShapeDtypeStruct((_NP, _H), _f32),
        ],
        interpret=interpret,
    )

    proj = pl.pallas_call(
        _proj_kernel,
        grid=(_GRID,),
        in_specs=[
            _nblk(_T - _ENC),
            _full((_H, 1)),
            _full((1, 1)),
        ],
        out_specs=pl.BlockSpec((_BN, _T - _ENC), lambda i: (i, 0)),
        out_shape=jax.ShapeDtypeStruct((_NP, _T - _ENC), _f32),
        interpret=interpret,
    )
    return embed, step, proj


_EMBED, _STEP, _PROJ = _make_tc_kernels()


def _pad_nodes(a):
    return jnp.pad(a, ((0, _NP - _N),) + ((0, 0),) * (a.ndim - 1))


def kernel(s_cat, k_cat, k_cont, o_cont, target, edge_index, params):
    spmm = _get_spmm()
    p = params

    s0 = _pad_nodes(s_cat[:, 0, :].astype(jnp.int32))
    kc = _pad_nodes(k_cat[:, :, 0].astype(jnp.int32))
    kc0 = _pad_nodes(k_cont[:, :, 0])
    kc1 = _pad_nodes(k_cont[:, :, 1])
    oc0 = _pad_nodes(o_cont[:, :, 0])
    oc1 = _pad_nodes(o_cont[:, :, 1])
    tg = _pad_nodes(target[:, :, 0])
    src = jnp.pad(edge_index[0], (0, _EPAD - _E),
                  constant_values=_NP - 1).astype(jnp.int32)
    dst = jnp.pad(edge_index[1], (0, _EPAD - _E),
                  constant_values=_NP - 1).astype(jnp.int32)
    src_r = src.reshape(_NW, _NCHUNK, _K)
    dst_r = dst.reshape(_NW, _NCHUNK, _K)

    ones_x = jnp.ones((_NP, _H), _f32)
    dgo = spmm(ones_x, dst_r, src_r)
    dgi = spmm(ones_x, src_r, dst_r)

    xs_hist, xs_fut, h0, hs0, norms = _EMBED(
        s0, kc, kc0, kc1, oc0, oc1, tg, dgo, dgi,
        p["s_cat_emb"][0], p["s_cat_emb"][1],
        p["static_W"], p["static_b"][None, :],
        p["k_cat_emb"][0],
        p["k_cont_vec"], p["k_cont_bias"],
        p["o_cont_vec"], p["o_cont_bias"],
        p["tgt_vec"], p["tgt_bias"],
        p["hist_down_W"], p["hist_down_b"][None, :],
        p["fut_down_W"], p["fut_down_b"][None, :])

    def run_layer(xs_list, h, hs, lp):
        wi, bi = lp["Wi"], lp["bi"][None, :]
        wh, bh = lp["Wh"], lp["bh"][None, :]
        ax_list = [spmm(x, src_r, dst_r) for x in xs_list]
        outs = []
        for ax in ax_list:
            ah = spmm(hs, src_r, dst_r)
            h, hs = _STEP(ax, ah, h, norms, wi, bi, wh, bh)
            outs.append((h, hs))
        return outs

    hist0 = run_layer([xs_hist[t] for t in range(_ENC)],
                      h0[0], hs0[0], p["hist_layers"][0])
    hist1 = run_layer([hs for _, hs in hist0],
                      h0[1], hs0[1], p["hist_layers"][1])
    fut0 = run_layer([xs_fut[t] for t in range(_T - _ENC)],
                     hist0[-1][0], hist0[-1][1], p["fut_layers"][0])
    fut1 = run_layer([hs for _, hs in fut0],
                     hist1[-1][0], hist1[-1][1], p["fut_layers"][1])

    hseq = jnp.stack([h for h, _ in fut1])
    y = _PROJ(hseq, p["out_W"], p["out_b"][None, :])
    return y[:_N, :, None]
